# Initial kernel scaffold; baseline (speedup 1.0000x reference)
import jax
import jax.numpy as jnp
from jax.experimental import pallas as pl


def kernel(embedding, W1, b1, W2, b2, edge_index):
    raise NotImplementedError("write your pallas kernel here")


# speedup vs baseline: 6.4051x; 6.4051x over previous
import functools

import jax
import jax.numpy as jnp
from jax import lax
from jax.experimental import pallas as pl
from jax.experimental.pallas import tpu as pltpu
from jax.experimental.pallas import tpu_sc as plsc

N = 10000
E = 160000
D = 256
H = 512

NC = 2
NS = 16
NW = NC * NS
EPW = E // NW
K = 40
NB = EPW // K
NPAD = 10240
RPT = NPAD // NS

_mesh = plsc.VectorSubcoreMesh(
    core_axis_name="c", subcore_axis_name="s", num_cores=NC, num_subcores=NS)

f32 = jnp.float32


def _deg_kernel_fn():
  @functools.partial(
      pl.kernel,
      out_type=jax.ShapeDtypeStruct((NC, NPAD, 128), f32),
      mesh=_mesh,
      scratch_types=[
          pltpu.VMEM((NB, K), jnp.int32),
          pltpu.VMEM((K, 128), f32),
          pltpu.VMEM_SHARED((NPAD, 128), f32),
      ],
  )
  def deg_kernel(dst_hbm, zeros_hbm, ones_hbm, out_hbm, dst_v, ones_v, acc):
    c = lax.axis_index("c")
    s = lax.axis_index("s")
    pltpu.sync_copy(dst_hbm.at[c, s], dst_v)
    pltpu.sync_copy(ones_hbm, ones_v)
    pltpu.sync_copy(zeros_hbm, acc.at[pl.ds(s * RPT, RPT)])
    plsc.subcore_barrier()

    def body(j, carry):
      pltpu.sync_copy(ones_v, acc.at[dst_v.at[j]], add=True)
      return carry

    lax.fori_loop(0, NB, body, 0)
    plsc.subcore_barrier()
    pltpu.sync_copy(acc.at[pl.ds(s * RPT, RPT)],
                    out_hbm.at[c, pl.ds(s * RPT, RPT)])

  return deg_kernel


def _scatter_kernel_fn(num_chunks):
  @functools.partial(
      pl.kernel,
      out_type=jax.ShapeDtypeStruct((NC, num_chunks, NPAD, 128), f32),
      mesh=_mesh,
      scratch_types=[
          pltpu.VMEM((NB, K), jnp.int32),
          pltpu.VMEM((NB, K), jnp.int32),
          pltpu.VMEM((K, 128), f32),
          pltpu.VMEM_SHARED((NPAD, 128), f32),
      ],
  )
  def scatter_kernel(m_hbm, src_hbm, dst_hbm, zeros_hbm, out_hbm,
                     src_v, dst_v, rows_v, acc):
    c = lax.axis_index("c")
    s = lax.axis_index("s")
    pltpu.sync_copy(src_hbm.at[c, s], src_v)
    pltpu.sync_copy(dst_hbm.at[c, s], dst_v)

    for cc in range(num_chunks):
      m_view = m_hbm.at[cc]
      pltpu.sync_copy(zeros_hbm, acc.at[pl.ds(s * RPT, RPT)])
      plsc.subcore_barrier()

      def body(j, carry):
        pltpu.sync_copy(m_view.at[src_v.at[j]], rows_v)
        pltpu.sync_copy(rows_v, acc.at[dst_v.at[j]], add=True)
        return carry

      lax.fori_loop(0, NB, body, 0)
      plsc.subcore_barrier()
      pltpu.sync_copy(acc.at[pl.ds(s * RPT, RPT)],
                      out_hbm.at[c, cc, pl.ds(s * RPT, RPT)])
      plsc.subcore_barrier()

  return scatter_kernel


_deg_call = _deg_kernel_fn()
_scatter4 = _scatter_kernel_fn(4)
_scatter2 = _scatter_kernel_fn(2)

RB = 1000


def _mm1_body(x_ref, w_ref, dinv_ref, o_ref):
  xs = x_ref[...] * dinv_ref[...]
  o_ref[0] = jnp.dot(xs, w_ref[...], preferred_element_type=f32)


def _mm2_body(hp_ref, m1_ref, dinv_ref, b1_ref, w2_ref, o_ref):
  k = pl.program_id(2)
  h = ((hp_ref[0, 0] + hp_ref[1, 0] + m1_ref[0]) * dinv_ref[...]
       + b1_ref[...][None, :])
  part = jnp.dot(h, w2_ref[...], preferred_element_type=f32)

  @pl.when(k == 0)
  def _():
    o_ref[0] = part

  @pl.when(k > 0)
  def _():
    o_ref[0] += part

  @pl.when(k == (H // 128) - 1)
  def _():
    o_ref[0] *= dinv_ref[...]


def _fin_body(gp_ref, m2_ref, dinv_ref, b2_ref, o_ref):
  o_ref[...] = ((gp_ref[0, 0] + gp_ref[1, 0] + m2_ref[0])
                * dinv_ref[...] + b2_ref[...][None, :])


def kernel(embedding, W1, b1, W2, b2, edge_index):
  src = edge_index[0].reshape(NC, NS, NB, K)
  dst = edge_index[1].reshape(NC, NS, NB, K)

  zeros_acc = jnp.zeros((RPT, 128), f32)
  ones_deg = jnp.ones((K, 128), f32)

  degp = _deg_call(dst, zeros_acc, ones_deg)
  deg = 1.0 + degp[0, :, 0] + degp[1, :, 0]
  dinv = lax.rsqrt(deg)
  dinv_c = dinv.reshape(NPAD, 1)

  m1 = pl.pallas_call(
      _mm1_body,
      grid=(N // RB, H // 128),
      in_specs=[
          pl.BlockSpec((RB, D), lambda i, c: (i, 0)),
          pl.BlockSpec((D, 128), lambda i, c: (0, c)),
          pl.BlockSpec((RB, 1), lambda i, c: (i, 0)),
      ],
      out_specs=pl.BlockSpec((1, RB, 128), lambda i, c: (c, i, 0)),
      out_shape=jax.ShapeDtypeStruct((H // 128, N, 128), f32),
  )(embedding, W1, dinv_c)

  acc1 = _scatter4(m1, src, dst, zeros_acc)

  m2 = pl.pallas_call(
      _mm2_body,
      grid=(N // RB, D // 128, H // 128),
      in_specs=[
          pl.BlockSpec((NC, 1, RB, 128), lambda i, c, k: (0, k, i, 0)),
          pl.BlockSpec((1, RB, 128), lambda i, c, k: (k, i, 0)),
          pl.BlockSpec((RB, 1), lambda i, c, k: (i, 0)),
          pl.BlockSpec((128,), lambda i, c, k: (k,)),
          pl.BlockSpec((128, 128), lambda i, c, k: (k, c)),
      ],
      out_specs=pl.BlockSpec((1, RB, 128), lambda i, c, k: (c, i, 0)),
      out_shape=jax.ShapeDtypeStruct((D // 128, N, 128), f32),
  )(acc1, m1, dinv_c, b1, W2)

  acc2 = _scatter2(m2, src, dst, zeros_acc)

  out = pl.pallas_call(
      _fin_body,
      grid=(N // RB, D // 128),
      in_specs=[
          pl.BlockSpec((NC, 1, RB, 128), lambda i, c: (0, c, i, 0)),
          pl.BlockSpec((1, RB, 128), lambda i, c: (c, i, 0)),
          pl.BlockSpec((RB, 1), lambda i, c: (i, 0)),
          pl.BlockSpec((128,), lambda i, c: (c,)),
      ],
      out_specs=pl.BlockSpec((RB, 128), lambda i, c: (i, c)),
      out_shape=jax.ShapeDtypeStruct((N, D), f32),
  )(acc2, m2, dinv_c, b2)

  return out


# speedup vs baseline: 9.6951x; 1.5136x over previous
import functools

import jax
import jax.numpy as jnp
from jax import lax
from jax.experimental import pallas as pl
from jax.experimental.pallas import tpu as pltpu
from jax.experimental.pallas import tpu_sc as plsc

N = 10000
E = 160000
D = 256
H = 512

NC = 2
NS = 16
NW = NC * NS
EPW = E // NW
K = 40
NB = EPW // K
NPAD = 10240
RPT = NPAD // NS

_mesh = plsc.VectorSubcoreMesh(
    core_axis_name="c", subcore_axis_name="s", num_cores=NC, num_subcores=NS)

f32 = jnp.float32


def _deg_kernel_fn():
  @functools.partial(
      pl.kernel,
      out_type=jax.ShapeDtypeStruct((NC, NPAD, 128), f32),
      mesh=_mesh,
      scratch_types=[
          pltpu.VMEM((NB, K), jnp.int32),
          pltpu.VMEM((K, 128), f32),
          pltpu.VMEM_SHARED((NPAD, 128), f32),
      ],
  )
  def deg_kernel(dst_hbm, zeros_hbm, ones_hbm, out_hbm, dst_v, ones_v, acc):
    c = lax.axis_index("c")
    s = lax.axis_index("s")
    pltpu.sync_copy(dst_hbm.at[c, s], dst_v)
    pltpu.sync_copy(ones_hbm, ones_v)
    pltpu.sync_copy(zeros_hbm, acc.at[pl.ds(s * RPT, RPT)])
    plsc.subcore_barrier()

    def body(j, carry):
      pltpu.sync_copy(ones_v, acc.at[dst_v.at[j]], add=True)
      return carry

    lax.fori_loop(0, NB, body, 0)
    plsc.subcore_barrier()
    pltpu.sync_copy(acc.at[pl.ds(s * RPT, RPT)],
                    out_hbm.at[c, pl.ds(s * RPT, RPT)])

  return deg_kernel


def _scatter_kernel_fn(num_chunks):
  @functools.partial(
      pl.kernel,
      out_type=jax.ShapeDtypeStruct((NC, num_chunks, NPAD, 128), f32),
      mesh=_mesh,
      scratch_types=[
          pltpu.VMEM((NB, K), jnp.int32),
          pltpu.VMEM((NB, K), jnp.int32),
          pltpu.VMEM((K, 128), f32),
          pltpu.VMEM((K, 128), f32),
          pltpu.VMEM_SHARED((NPAD, 128), f32),
          pltpu.SemaphoreType.DMA,
          pltpu.SemaphoreType.DMA,
      ],
  )
  def scatter_kernel(m_hbm, src_hbm, dst_hbm, zeros_hbm, out_hbm,
                     src_v, dst_v, rows_a, rows_b, acc, sem_a, sem_b):
    c = lax.axis_index("c")
    s = lax.axis_index("s")
    pltpu.sync_copy(src_hbm.at[c, s], src_v)
    pltpu.sync_copy(dst_hbm.at[c, s], dst_v)

    for cc in range(num_chunks):
      m_view = m_hbm.at[cc]
      pltpu.sync_copy(zeros_hbm, acc.at[pl.ds(s * RPT, RPT)])
      plsc.subcore_barrier()

      def start_g(j, buf, sem):
        pltpu.async_copy(m_view.at[src_v.at[j]], buf, sem)

      def finish_g(j, buf, sem):
        pltpu.make_async_copy(m_view.at[src_v.at[j]], buf, sem).wait()

      def step(j, buf, sem):
        finish_g(j, buf, sem)
        pltpu.sync_copy(buf, acc.at[dst_v.at[j]], add=True)

      start_g(0, rows_a, sem_a)

      def body(j, carry):
        @pl.when(j % 2 == 0)
        def _():
          @pl.when(j + 1 < NB)
          def _():
            start_g(j + 1, rows_b, sem_b)
          step(j, rows_a, sem_a)

        @pl.when(j % 2 == 1)
        def _():
          @pl.when(j + 1 < NB)
          def _():
            start_g(j + 1, rows_a, sem_a)
          step(j, rows_b, sem_b)

        return carry

      lax.fori_loop(0, NB, body, 0)
      plsc.subcore_barrier()
      pltpu.sync_copy(acc.at[pl.ds(s * RPT, RPT)],
                      out_hbm.at[c, cc, pl.ds(s * RPT, RPT)])
      plsc.subcore_barrier()

  return scatter_kernel


_deg_call = _deg_kernel_fn()
_scatter4 = _scatter_kernel_fn(4)
_scatter2 = _scatter_kernel_fn(2)

RB = 1000


def _mm1_body(x_ref, w_ref, dinv_ref, o_ref):
  xs = x_ref[...] * dinv_ref[...]
  o_ref[0] = jnp.dot(xs, w_ref[...], preferred_element_type=f32)


def _mm2_body(hp_ref, m1_ref, dinv_ref, b1_ref, w2_ref, o_ref):
  k = pl.program_id(2)
  h = ((hp_ref[0, 0] + hp_ref[1, 0] + m1_ref[0]) * dinv_ref[...]
       + b1_ref[...][None, :])
  part = jnp.dot(h, w2_ref[...], preferred_element_type=f32)

  @pl.when(k == 0)
  def _():
    o_ref[0] = part

  @pl.when(k > 0)
  def _():
    o_ref[0] += part

  @pl.when(k == (H // 128) - 1)
  def _():
    o_ref[0] *= dinv_ref[...]


def _fin_body(gp_ref, m2_ref, dinv_ref, b2_ref, o_ref):
  o_ref[...] = ((gp_ref[0, 0] + gp_ref[1, 0] + m2_ref[0])
                * dinv_ref[...] + b2_ref[...][None, :])


def kernel(embedding, W1, b1, W2, b2, edge_index):
  src = edge_index[0].reshape(NC, NS, NB, K)
  dst = edge_index[1].reshape(NC, NS, NB, K)

  zeros_acc = jnp.zeros((RPT, 128), f32)
  ones_deg = jnp.ones((K, 128), f32)

  degp = _deg_call(dst, zeros_acc, ones_deg)
  deg = 1.0 + degp[0, :, 0] + degp[1, :, 0]
  dinv = lax.rsqrt(deg)
  dinv_c = dinv.reshape(NPAD, 1)

  m1 = pl.pallas_call(
      _mm1_body,
      grid=(N // RB, H // 128),
      in_specs=[
          pl.BlockSpec((RB, D), lambda i, c: (i, 0)),
          pl.BlockSpec((D, 128), lambda i, c: (0, c)),
          pl.BlockSpec((RB, 1), lambda i, c: (i, 0)),
      ],
      out_specs=pl.BlockSpec((1, RB, 128), lambda i, c: (c, i, 0)),
      out_shape=jax.ShapeDtypeStruct((H // 128, N, 128), f32),
  )(embedding, W1, dinv_c)

  acc1 = _scatter4(m1, src, dst, zeros_acc)

  m2 = pl.pallas_call(
      _mm2_body,
      grid=(N // RB, D // 128, H // 128),
      in_specs=[
          pl.BlockSpec((NC, 1, RB, 128), lambda i, c, k: (0, k, i, 0)),
          pl.BlockSpec((1, RB, 128), lambda i, c, k: (k, i, 0)),
          pl.BlockSpec((RB, 1), lambda i, c, k: (i, 0)),
          pl.BlockSpec((128,), lambda i, c, k: (k,)),
          pl.BlockSpec((128, 128), lambda i, c, k: (k, c)),
      ],
      out_specs=pl.BlockSpec((1, RB, 128), lambda i, c, k: (c, i, 0)),
      out_shape=jax.ShapeDtypeStruct((D // 128, N, 128), f32),
  )(acc1, m1, dinv_c, b1, W2)

  acc2 = _scatter2(m2, src, dst, zeros_acc)

  out = pl.pallas_call(
      _fin_body,
      grid=(N // RB, D // 128),
      in_specs=[
          pl.BlockSpec((NC, 1, RB, 128), lambda i, c: (0, c, i, 0)),
          pl.BlockSpec((1, RB, 128), lambda i, c: (c, i, 0)),
          pl.BlockSpec((RB, 1), lambda i, c: (i, 0)),
          pl.BlockSpec((128,), lambda i, c: (c,)),
      ],
      out_specs=pl.BlockSpec((RB, 128), lambda i, c: (i, c)),
      out_shape=jax.ShapeDtypeStruct((N, D), f32),
  )(acc2, m2, dinv_c, b2)

  return out
